# Initial kernel scaffold; baseline (speedup 1.0000x reference)
import jax
import jax.numpy as jnp
from jax.experimental import pallas as pl


def kernel(x, edge_index, edge_attr, x_emb1, x_emb2, ee1_a, ee2_a, W1_a, b1_a, W2_a, b2_a, bn1_g, bn1_b, ee1_b, ee2_b, W1_b, b1_b, W2_b, b2_b, bn2_g, bn2_b):
    raise NotImplementedError("write your pallas kernel here")


# speedup vs baseline: 6.6892x; 6.6892x over previous
import functools

import jax
import jax.numpy as jnp
from jax import lax
from jax.experimental import pallas as pl
from jax.experimental.pallas import tpu as pltpu
from jax.experimental.pallas import tpu_sc as plsc

_N = 10000
_E = 320000
_D = 128
_NC = 2
_NS = 16
_NW = _NC * _NS
_EW = _E // _NW
_CH = 80
_GC = 5
_NG = _EW // (_GC * _CH)
_RC = 80
_NRCH = _N // _RC
_EBR = 24
_BLK = 1000
_NB = _N // _BLK
_LOOP_ROW = 4 * 3 + 0

_mesh = plsc.VectorSubcoreMesh(core_axis_name="c", subcore_axis_name="s")


@functools.partial(
    pl.kernel,
    out_type=(
        jax.ShapeDtypeStruct((_N, _D), jnp.float32),
        jax.ShapeDtypeStruct((_EBR, _D), jnp.float32),
        jax.ShapeDtypeStruct((_EBR, _D), jnp.float32),
    ),
    mesh=_mesh,
    scratch_types=[
        pltpu.VMEM((_CH,), jnp.int32),
        pltpu.VMEM((_CH,), jnp.int32),
        pltpu.VMEM((_CH, _D), jnp.float32),
        pltpu.VMEM((_CH, _D), jnp.float32),
        pltpu.VMEM((6, _D), jnp.float32),
        pltpu.VMEM((3, _D), jnp.float32),
        pltpu.VMEM((_EBR, _D), jnp.float32),
        pltpu.SemaphoreType.DMA,
        pltpu.SemaphoreType.DMA,
    ],
)
def _prep_sc(x0_hbm, x1_hbm, emb1_hbm, emb2_hbm, e1a_hbm, e2a_hbm,
             e1b_hbm, e2b_hbm, h0_out, ebta_out, ebtb_out,
             idx0_v, idx1_v, rows_a, rows_b, t6, t3, tebt, sem0, sem1):
    c = lax.axis_index("c")
    s = lax.axis_index("s")
    wid = s * _NC + c
    n_chunks = _N // _CH

    for i in range((n_chunks + _NW - 1) // _NW):
        ci = wid + _NW * i

        @pl.when(ci < n_chunks)
        def _():
            base = ci * _CH
            pltpu.sync_copy(x0_hbm.at[pl.ds(base, _CH)], idx0_v)
            pltpu.sync_copy(x1_hbm.at[pl.ds(base, _CH)], idx1_v)
            ga = pltpu.async_copy(emb1_hbm.at[idx0_v], rows_a, sem0)
            gb = pltpu.async_copy(emb2_hbm.at[idx1_v], rows_b, sem1)
            ga.wait()
            gb.wait()

            def add_row(r, carry):
                for k in range(_D // 16):
                    sl = pl.ds(k * 16, 16)
                    rows_a[r, sl] = rows_a[r, sl] + rows_b[r, sl]
                return carry

            lax.fori_loop(0, _CH, add_row, 0)
            pltpu.sync_copy(rows_a, h0_out.at[pl.ds(base, _CH)])

    @pl.when(wid == 0)
    def _():
        for src1, src2, out in ((e1a_hbm, e2a_hbm, ebta_out),
                                (e1b_hbm, e2b_hbm, ebtb_out)):
            pltpu.sync_copy(src1, t6)
            pltpu.sync_copy(src2, t3)
            for i in range(6):
                for j in range(3):
                    for k in range(_D // 16):
                        sl = pl.ds(k * 16, 16)
                        tebt[i * 3 + j, sl] = t6[i, sl] + t3[j, sl]
            pltpu.sync_copy(tebt, out)


@functools.partial(
    pl.kernel,
    out_type=jax.ShapeDtypeStruct((_NC, _N, _D), jnp.float32),
    mesh=_mesh,
    scratch_types=[
        pltpu.VMEM((_GC, _CH), jnp.int32),
        pltpu.VMEM((_GC, _CH), jnp.int32),
        pltpu.VMEM((_GC, _CH), jnp.int32),
        pltpu.VMEM((_GC, _CH), jnp.int32),
        pltpu.VMEM((_CH, _D), jnp.float32),
        pltpu.VMEM((_CH, _D), jnp.float32),
        pltpu.VMEM_SHARED((_N, _D), jnp.float32),
        pltpu.VMEM_SHARED((_EBR, _D), jnp.float32),
        pltpu.SemaphoreType.DMA,
        pltpu.SemaphoreType.DMA,
    ],
)
def _conv_sc(h_hbm, src4, dst4, ea04, ea14, ebt_hbm, z_hbm, parts_out,
             src_v, dst_v, ea0_v, ea1_v, rows_a, rows_b,
             agg_sh, ebt_sh, sem0, sem1):
    c = lax.axis_index("c")
    s = lax.axis_index("s")
    wid = s * _NC + c

    pltpu.sync_copy(z_hbm, rows_a)
    for i in range((_NRCH + _NS - 1) // _NS):
        rc = s + _NS * i

        @pl.when(rc < _NRCH)
        def _():
            pltpu.sync_copy(rows_a, agg_sh.at[pl.ds(rc * _RC, _RC)])

    @pl.when(s == 0)
    def _():
        pltpu.sync_copy(ebt_hbm, rows_b.at[pl.ds(0, _EBR)])
        pltpu.sync_copy(rows_b.at[pl.ds(0, _EBR)], ebt_sh)

    plsc.subcore_barrier()

    def group(g, carry):
        pltpu.sync_copy(src4.at[wid, g], src_v)
        pltpu.sync_copy(dst4.at[wid, g], dst_v)
        pltpu.sync_copy(ea04.at[wid, g], ea0_v)
        pltpu.sync_copy(ea14.at[wid, g], ea1_v)

        def xform(j, cr):
            for k in range(_CH // 16):
                sl = pl.ds(k * 16, 16)
                ea0_v[j, sl] = ea0_v[j, sl] * 3 + ea1_v[j, sl]
            return cr

        lax.fori_loop(0, _GC, xform, 0)

        def edge_chunk(j, cr):
            ga = pltpu.async_copy(h_hbm.at[src_v.at[j]], rows_a, sem0)
            gb = pltpu.async_copy(ebt_sh.at[ea0_v.at[j]], rows_b, sem1)
            ga.wait()
            gb.wait()
            pltpu.sync_copy(rows_a, agg_sh.at[dst_v.at[j]], add=True)
            pltpu.sync_copy(rows_b, agg_sh.at[dst_v.at[j]], add=True)
            return cr

        lax.fori_loop(0, _GC, edge_chunk, 0)
        return carry

    lax.fori_loop(0, _NG, group, 0)

    plsc.subcore_barrier()

    for i in range((_NRCH + _NS - 1) // _NS):
        rc = s + _NS * i

        @pl.when(rc < _NRCH)
        def _():
            sl = pl.ds(rc * _RC, _RC)
            pltpu.sync_copy(agg_sh.at[sl], rows_a)
            pltpu.sync_copy(rows_a, parts_out.at[c, sl])


def _mlp_body(p_ref, h_ref, ebt_ref, w1_ref, b1_ref, w2_ref, b2_ref,
              y_ref, st_ref, acc_ref):
    i = pl.program_id(0)
    agg = (p_ref[0] + p_ref[1] + h_ref[...]
           + ebt_ref[_LOOP_ROW:_LOOP_ROW + 1, :])
    hmid = jnp.maximum(
        jnp.dot(agg, w1_ref[...], preferred_element_type=jnp.float32)
        + b1_ref[...], 0.0)
    y = jnp.maximum(
        jnp.dot(hmid, w2_ref[...], preferred_element_type=jnp.float32)
        + b2_ref[...], 0.0)
    y_ref[...] = y
    blk = jnp.concatenate(
        [jnp.sum(y, axis=0, keepdims=True),
         jnp.sum(y * y, axis=0, keepdims=True)], axis=0)

    @pl.when(i == 0)
    def _():
        acc_ref[...] = blk

    @pl.when(i > 0)
    def _():
        acc_ref[...] = acc_ref[...] + blk

    st_ref[...] = acc_ref[...]


_mlp_tc = pl.pallas_call(
    _mlp_body,
    grid=(_NB,),
    in_specs=[
        pl.BlockSpec((_NC, _BLK, _D), lambda i: (0, i, 0)),
        pl.BlockSpec((_BLK, _D), lambda i: (i, 0)),
        pl.BlockSpec((_EBR, _D), lambda i: (0, 0)),
        pl.BlockSpec((_D, 2 * _D), lambda i: (0, 0)),
        pl.BlockSpec((1, 2 * _D), lambda i: (0, 0)),
        pl.BlockSpec((2 * _D, _D), lambda i: (0, 0)),
        pl.BlockSpec((1, _D), lambda i: (0, 0)),
    ],
    out_specs=[
        pl.BlockSpec((_BLK, _D), lambda i: (i, 0)),
        pl.BlockSpec((2, _D), lambda i: (0, 0)),
    ],
    out_shape=[
        jax.ShapeDtypeStruct((_N, _D), jnp.float32),
        jax.ShapeDtypeStruct((2, _D), jnp.float32),
    ],
    scratch_shapes=[pltpu.VMEM((2, _D), jnp.float32)],
)


def _norm_body(y_ref, st_ref, g_ref, b_ref, o_ref):
    mu = st_ref[0:1, :] * (1.0 / _N)
    ex2 = st_ref[1:2, :] * (1.0 / _N)
    var = ex2 - mu * mu
    scale = g_ref[...] * lax.rsqrt(var + 1e-5)
    o_ref[...] = (y_ref[...] - mu) * scale + b_ref[...]


_norm_tc = pl.pallas_call(
    _norm_body,
    grid=(_NB,),
    in_specs=[
        pl.BlockSpec((_BLK, _D), lambda i: (i, 0)),
        pl.BlockSpec((2, _D), lambda i: (0, 0)),
        pl.BlockSpec((1, _D), lambda i: (0, 0)),
        pl.BlockSpec((1, _D), lambda i: (0, 0)),
    ],
    out_specs=pl.BlockSpec((_BLK, _D), lambda i: (i, 0)),
    out_shape=jax.ShapeDtypeStruct((_N, _D), jnp.float32),
)


def kernel(x, edge_index, edge_attr, x_emb1, x_emb2,
           ee1_a, ee2_a, W1_a, b1_a, W2_a, b2_a, bn1_g, bn1_b,
           ee1_b, ee2_b, W1_b, b1_b, W2_b, b2_b, bn2_g, bn2_b):
    i32 = jnp.int32
    x0 = x[:, 0].astype(i32)
    x1 = x[:, 1].astype(i32)
    eshape = (_NW, _NG, _GC, _CH)
    src = edge_index[0].astype(i32).reshape(eshape)
    dst = edge_index[1].astype(i32).reshape(eshape)
    ea0 = edge_attr[:, 0].astype(i32).reshape(eshape)
    ea1 = edge_attr[:, 1].astype(i32).reshape(eshape)
    zrows = jnp.zeros((_RC, _D), jnp.float32)
    b1a = b1_a.reshape(1, -1)
    b2a = b2_a.reshape(1, -1)
    b1b = b1_b.reshape(1, -1)
    b2b = b2_b.reshape(1, -1)

    h0, ebta, ebtb = _prep_sc(x0, x1, x_emb1, x_emb2,
                              ee1_a, ee2_a, ee1_b, ee2_b)

    parts1 = _conv_sc(h0, src, dst, ea0, ea1, ebta, zrows)
    y1, st1 = _mlp_tc(parts1, h0, ebta, W1_a, b1a, W2_a, b2a)
    h1 = _norm_tc(y1, st1, bn1_g.reshape(1, -1), bn1_b.reshape(1, -1))

    parts2 = _conv_sc(h1, src, dst, ea0, ea1, ebtb, zrows)
    y2, st2 = _mlp_tc(parts2, h1, ebtb, W1_b, b1b, W2_b, b2b)
    h2 = _norm_tc(y2, st2, bn2_g.reshape(1, -1), bn2_b.reshape(1, -1))
    return h2
